# baseline (speedup 1.0000x reference)
import functools

import jax
import jax.numpy as jnp
from jax import lax
from jax.experimental import pallas as pl
from jax.experimental.pallas import tpu as pltpu
from jax.experimental.pallas import tpu_sc as plsc

NQ = 4
CHUNK = 128
LANES = 16
TBLK = 512


def _transpose_table(tab_t):
    dim, vocab = tab_t.shape
    grid = (vocab + TBLK - 1) // TBLK

    def body(i_ref, o_ref):
        o_ref[...] = i_ref[...].T

    return pl.pallas_call(
        body,
        grid=(grid,),
        in_specs=[pl.BlockSpec((dim, TBLK), lambda i: (0, i))],
        out_specs=pl.BlockSpec((TBLK, dim), lambda i: (i, 0)),
        out_shape=jax.ShapeDtypeStruct((vocab, dim), jnp.float32),
    )(tab_t)


def _make_sc_kernel(l_seq, b, dim, nc, ns):
    nw = nc * ns
    assert b == nw * CHUNK

    mesh = plsc.VectorSubcoreMesh(core_axis_name="c", subcore_axis_name="s")

    @functools.partial(
        pl.kernel,
        out_type=jax.ShapeDtypeStruct((l_seq, b, dim * NQ), jnp.float32),
        mesh=mesh,
        compiler_params=pltpu.CompilerParams(needs_layout_passes=False,
                                             use_tc_tiling_on_sc=False),
        scratch_types=[
            pltpu.VMEM((l_seq, CHUNK), jnp.int32),
            pltpu.VMEM((CHUNK,), jnp.int32),
            pltpu.VMEM((CHUNK, dim), jnp.float32),
            pltpu.VMEM((CHUNK, dim), jnp.float32),
            pltpu.VMEM((CHUNK, dim), jnp.float32),
            pltpu.VMEM((CHUNK, dim), jnp.float32),
            pltpu.VMEM((CHUNK, dim * NQ), jnp.float32),
            pltpu.SemaphoreType.DMA,
        ],
    )
    def qembed(xt_ref, s_ref, vi_ref, vj_ref, vk_ref, out_ref,
               idx_v, idxg_v, g0, g1, g2, g3, o_v, sem):
        wid = lax.axis_index("c") * ns + lax.axis_index("s")
        col0 = wid * CHUNK
        pltpu.sync_copy(xt_ref.at[:, pl.ds(col0, CHUNK)], idx_v)

        iota = lax.iota(jnp.int32, LANES)
        pats = [[iota * NQ + (h * LANES * NQ + q) for h in range(dim // LANES)]
                for q in range(NQ)]
        gbufs = (g0, g1, g2, g3)
        tables = (s_ref, vi_ref, vj_ref, vk_ref)
        nj = CHUNK // LANES

        def do_chunk(t, carry):
            for j in range(nj):
                idxg_v[pl.ds(j * LANES, LANES)] = idx_v[t, pl.ds(j * LANES, LANES)]
            cps = [pltpu.async_copy(tables[q].at[idxg_v], gbufs[q], sem)
                   for q in range(NQ)]
            for cp in cps:
                cp.wait()

            def interleave(i, carry2):
                row = jnp.full((LANES,), 0, jnp.int32) + i
                for q in range(NQ):
                    for h in range(dim // LANES):
                        vals = gbufs[q][i, pl.ds(h * LANES, LANES)]
                        plsc.store_scatter(o_v, [row, pats[q][h]], vals)
                return carry2

            lax.fori_loop(0, CHUNK, interleave, 0, unroll=2)
            pltpu.sync_copy(o_v, out_ref.at[t, pl.ds(col0, CHUNK), :])
            return carry

        lax.fori_loop(0, l_seq, do_chunk, 0)

    return qembed


@jax.jit
def kernel(x, scalar, vector_i, vector_j, vector_k):
    b, l_seq = x.shape
    vocab, dim = scalar.shape
    info = plsc.get_sparse_core_info()
    k = _make_sc_kernel(l_seq, b, dim, info.num_cores, info.num_subcores)
    tabs = [_transpose_table(t.T)
            for t in (scalar, vector_i, vector_j, vector_k)]
    out = k(x.T.astype(jnp.int32), *tabs)
    return out.reshape(l_seq, b, dim, NQ).transpose(1, 0, 2, 3)


# speedup vs baseline: 2.4578x; 2.4578x over previous
import functools

import jax
import jax.numpy as jnp
from jax import lax
from jax.experimental import pallas as pl
from jax.experimental.pallas import tpu as pltpu
from jax.experimental.pallas import tpu_sc as plsc

NQ = 4
CHUNK = 128
LANES = 16
TBLK = 16384


def _transpose_table(tab_t):
    dim, vocab = tab_t.shape
    grid = (vocab + TBLK - 1) // TBLK

    def body(i_ref, o_ref):
        o_ref[...] = i_ref[...].T

    return pl.pallas_call(
        body,
        grid=(grid,),
        in_specs=[pl.BlockSpec((dim, TBLK), lambda i: (0, i))],
        out_specs=pl.BlockSpec((TBLK, dim), lambda i: (i, 0)),
        out_shape=jax.ShapeDtypeStruct((vocab, dim), jnp.float32),
    )(tab_t)


def _make_sc_kernel(l_seq, b, dim, nc, ns):
    nw = nc * ns
    assert b == nw * CHUNK

    mesh = plsc.VectorSubcoreMesh(core_axis_name="c", subcore_axis_name="s")

    @functools.partial(
        pl.kernel,
        out_type=jax.ShapeDtypeStruct((l_seq, b, dim * NQ), jnp.float32),
        mesh=mesh,
        compiler_params=pltpu.CompilerParams(needs_layout_passes=False,
                                             use_tc_tiling_on_sc=False),
        scratch_types=[
            pltpu.VMEM((l_seq, CHUNK), jnp.int32),
            pltpu.VMEM((CHUNK,), jnp.int32),
            pltpu.VMEM((CHUNK, dim), jnp.float32),
            pltpu.VMEM((CHUNK, dim), jnp.float32),
            pltpu.VMEM((CHUNK, dim), jnp.float32),
            pltpu.VMEM((CHUNK, dim), jnp.float32),
            pltpu.VMEM((CHUNK, dim * NQ), jnp.float32),
            pltpu.SemaphoreType.DMA,
        ],
    )
    def qembed(xt_ref, s_ref, vi_ref, vj_ref, vk_ref, out_ref,
               idx_v, idxg_v, g0, g1, g2, g3, o_v, sem):
        wid = lax.axis_index("c") * ns + lax.axis_index("s")
        col0 = wid * CHUNK
        pltpu.sync_copy(xt_ref.at[:, pl.ds(col0, CHUNK)], idx_v)

        iota = lax.iota(jnp.int32, LANES)
        pats = [[iota * NQ + (h * LANES * NQ + q) for h in range(dim // LANES)]
                for q in range(NQ)]
        gbufs = (g0, g1, g2, g3)
        tables = (s_ref, vi_ref, vj_ref, vk_ref)
        nj = CHUNK // LANES

        def do_chunk(t, carry):
            for j in range(nj):
                idxg_v[pl.ds(j * LANES, LANES)] = idx_v[t, pl.ds(j * LANES, LANES)]
            cps = [pltpu.async_copy(tables[q].at[idxg_v], gbufs[q], sem)
                   for q in range(NQ)]
            for cp in cps:
                cp.wait()

            def interleave(i, carry2):
                row = jnp.full((LANES,), 0, jnp.int32) + i
                for q in range(NQ):
                    for h in range(dim // LANES):
                        vals = gbufs[q][i, pl.ds(h * LANES, LANES)]
                        plsc.store_scatter(o_v, [row, pats[q][h]], vals)
                return carry2

            lax.fori_loop(0, CHUNK, interleave, 0, unroll=2)
            pltpu.sync_copy(o_v, out_ref.at[t, pl.ds(col0, CHUNK), :])
            return carry

        lax.fori_loop(0, l_seq, do_chunk, 0)

    return qembed


@jax.jit
def kernel(x, scalar, vector_i, vector_j, vector_k):
    b, l_seq = x.shape
    vocab, dim = scalar.shape
    info = plsc.get_sparse_core_info()
    k = _make_sc_kernel(l_seq, b, dim, info.num_cores, info.num_subcores)
    tabs = [_transpose_table(t.T)
            for t in (scalar, vector_i, vector_j, vector_k)]
    out = k(x.T.astype(jnp.int32), *tabs)
    return out.reshape(l_seq, b, dim, NQ).transpose(1, 0, 2, 3)


# speedup vs baseline: 2.4662x; 1.0034x over previous
import functools

import jax
import jax.numpy as jnp
from jax import lax
from jax.experimental import pallas as pl
from jax.experimental.pallas import tpu as pltpu
from jax.experimental.pallas import tpu_sc as plsc

NQ = 4
CHUNK = 128
LANES = 16
TBLK = 16384


def _transpose_table(tab_t):
    dim, vocab = tab_t.shape
    grid = (vocab + TBLK - 1) // TBLK

    def body(i_ref, o_ref):
        eye = jnp.where(
            lax.broadcasted_iota(jnp.int32, (128, 128), 0)
            == lax.broadcasted_iota(jnp.int32, (128, 128), 1),
            jnp.float32(1.0), jnp.float32(0.0))
        x = i_ref[...]
        for c in range(TBLK // 128):
            o_ref[pl.ds(c * 128, 128), :] = lax.dot_general(
                eye, x[:, c * 128:(c + 1) * 128],
                (((1,), (1,)), ((), ())),
                preferred_element_type=jnp.float32)

    return pl.pallas_call(
        body,
        grid=(grid,),
        in_specs=[pl.BlockSpec((dim, TBLK), lambda i: (0, i))],
        out_specs=pl.BlockSpec((TBLK, dim), lambda i: (i, 0)),
        out_shape=jax.ShapeDtypeStruct((vocab, dim), jnp.float32),
    )(tab_t)


def _make_sc_kernel(l_seq, b, dim, nc, ns):
    nw = nc * ns
    assert b == nw * CHUNK

    mesh = plsc.VectorSubcoreMesh(core_axis_name="c", subcore_axis_name="s")

    @functools.partial(
        pl.kernel,
        out_type=jax.ShapeDtypeStruct((l_seq, b, dim * NQ), jnp.float32),
        mesh=mesh,
        compiler_params=pltpu.CompilerParams(needs_layout_passes=False,
                                             use_tc_tiling_on_sc=False),
        scratch_types=[
            pltpu.VMEM((l_seq, CHUNK), jnp.int32),
            pltpu.VMEM((CHUNK,), jnp.int32),
            pltpu.VMEM((CHUNK, dim), jnp.float32),
            pltpu.VMEM((CHUNK, dim), jnp.float32),
            pltpu.VMEM((CHUNK, dim), jnp.float32),
            pltpu.VMEM((CHUNK, dim), jnp.float32),
            pltpu.VMEM((CHUNK, dim * NQ), jnp.float32),
            pltpu.SemaphoreType.DMA,
        ],
    )
    def qembed(xt_ref, s_ref, vi_ref, vj_ref, vk_ref, out_ref,
               idx_v, idxg_v, g0, g1, g2, g3, o_v, sem):
        wid = lax.axis_index("c") * ns + lax.axis_index("s")
        col0 = wid * CHUNK
        pltpu.sync_copy(xt_ref.at[:, pl.ds(col0, CHUNK)], idx_v)

        iota = lax.iota(jnp.int32, LANES)
        pats = [[iota * NQ + (h * LANES * NQ + q) for h in range(dim // LANES)]
                for q in range(NQ)]
        gbufs = (g0, g1, g2, g3)
        tables = (s_ref, vi_ref, vj_ref, vk_ref)
        nj = CHUNK // LANES

        def do_chunk(t, carry):
            for j in range(nj):
                idxg_v[pl.ds(j * LANES, LANES)] = idx_v[t, pl.ds(j * LANES, LANES)]
            cps = [pltpu.async_copy(tables[q].at[idxg_v], gbufs[q], sem)
                   for q in range(NQ)]
            for cp in cps:
                cp.wait()

            def interleave(i, carry2):
                row = jnp.full((LANES,), 0, jnp.int32) + i
                for q in range(NQ):
                    for h in range(dim // LANES):
                        vals = gbufs[q][i, pl.ds(h * LANES, LANES)]
                        plsc.store_scatter(o_v, [row, pats[q][h]], vals)
                return carry2

            lax.fori_loop(0, CHUNK, interleave, 0, unroll=2)
            pltpu.sync_copy(o_v, out_ref.at[t, pl.ds(col0, CHUNK), :])
            return carry

        lax.fori_loop(0, l_seq, do_chunk, 0)

    return qembed


@jax.jit
def kernel(x, scalar, vector_i, vector_j, vector_k):
    b, l_seq = x.shape
    vocab, dim = scalar.shape
    info = plsc.get_sparse_core_info()
    k = _make_sc_kernel(l_seq, b, dim, info.num_cores, info.num_subcores)
    tabs = [_transpose_table(t.T)
            for t in (scalar, vector_i, vector_j, vector_k)]
    out = k(x.T.astype(jnp.int32), *tabs)
    return out.reshape(l_seq, b, dim, NQ).transpose(1, 0, 2, 3)


# speedup vs baseline: 6.1319x; 2.4864x over previous
import functools

import jax
import jax.numpy as jnp
from jax import lax
from jax.experimental import pallas as pl
from jax.experimental.pallas import tpu as pltpu
from jax.experimental.pallas import tpu_sc as plsc

NQ = 4
CHUNK = 128
LANES = 16
TBLK = 8192


def _fuse_tables(tabs_t):
    dim, vocab = tabs_t[0].shape
    grid = (vocab + TBLK - 1) // TBLK

    def body(t0_ref, t1_ref, t2_ref, t3_ref, o_ref):
        eye = jnp.where(
            lax.broadcasted_iota(jnp.int32, (128, 128), 0)
            == lax.broadcasted_iota(jnp.int32, (128, 128), 1),
            jnp.float32(1.0), jnp.float32(0.0))
        for q, t_ref in enumerate((t0_ref, t1_ref, t2_ref, t3_ref)):
            x = t_ref[...]
            for c in range(TBLK // 128):
                o_ref[pl.ds(c * 128, 128), pl.ds(q * dim, dim)] = (
                    lax.dot_general(eye, x[:, c * 128:(c + 1) * 128],
                                    (((1,), (1,)), ((), ())),
                                    preferred_element_type=jnp.float32))

    in_spec = pl.BlockSpec((dim, TBLK), lambda i: (0, i))
    return pl.pallas_call(
        body,
        grid=(grid,),
        in_specs=[in_spec, in_spec, in_spec, in_spec],
        out_specs=pl.BlockSpec((TBLK, NQ * dim), lambda i: (i, 0)),
        out_shape=jax.ShapeDtypeStruct((vocab, NQ * dim), jnp.float32),
    )(*tabs_t)


def _make_sc_kernel(l_seq, b, dim, nc, ns):
    nw = nc * ns
    assert b == nw * CHUNK

    mesh = plsc.VectorSubcoreMesh(core_axis_name="c", subcore_axis_name="s")

    @functools.partial(
        pl.kernel,
        out_type=jax.ShapeDtypeStruct((l_seq, b, dim * NQ), jnp.float32),
        mesh=mesh,
        compiler_params=pltpu.CompilerParams(needs_layout_passes=False),
        scratch_types=[
            pltpu.VMEM((l_seq, CHUNK), jnp.int32),
            pltpu.VMEM((CHUNK,), jnp.int32),
            pltpu.VMEM((CHUNK, NQ * dim), jnp.float32),
            pltpu.VMEM((CHUNK, dim * NQ), jnp.float32),
            pltpu.SemaphoreType.DMA,
        ],
    )
    def qembed(xt_ref, tab_ref, out_ref, idx_v, idxg_v, g_v, o_v, sem):
        wid = lax.axis_index("c") * ns + lax.axis_index("s")
        col0 = wid * CHUNK
        pltpu.sync_copy(xt_ref.at[:, pl.ds(col0, CHUNK)], idx_v)

        iota = lax.iota(jnp.int32, LANES)
        pats = [[iota * NQ + (h * LANES * NQ + q) for h in range(dim // LANES)]
                for q in range(NQ)]
        nj = CHUNK // LANES

        def do_chunk(t, carry):
            for j in range(nj):
                idxg_v[pl.ds(j * LANES, LANES)] = idx_v[t, pl.ds(j * LANES, LANES)]
            pltpu.async_copy(tab_ref.at[idxg_v], g_v, sem).wait()

            def interleave(i, carry2):
                row = jnp.full((LANES,), 0, jnp.int32) + i
                for q in range(NQ):
                    for h in range(dim // LANES):
                        vals = g_v[i, pl.ds(q * dim + h * LANES, LANES)]
                        plsc.store_scatter(o_v, [row, pats[q][h]], vals)
                return carry2

            lax.fori_loop(0, CHUNK, interleave, 0, unroll=2)
            pltpu.sync_copy(o_v, out_ref.at[t, pl.ds(col0, CHUNK), :])
            return carry

        lax.fori_loop(0, l_seq, do_chunk, 0)

    return qembed


@jax.jit
def kernel(x, scalar, vector_i, vector_j, vector_k):
    b, l_seq = x.shape
    vocab, dim = scalar.shape
    info = plsc.get_sparse_core_info()
    k = _make_sc_kernel(l_seq, b, dim, info.num_cores, info.num_subcores)
    fused = _fuse_tables([t.T for t in (scalar, vector_i, vector_j, vector_k)])
    out = k(x.T.astype(jnp.int32), fused)
    return out.reshape(l_seq, b, dim, NQ).transpose(1, 0, 2, 3)


# speedup vs baseline: 6.5845x; 1.0738x over previous
import functools

import jax
import jax.numpy as jnp
from jax import lax
from jax.experimental import pallas as pl
from jax.experimental.pallas import tpu as pltpu
from jax.experimental.pallas import tpu_sc as plsc

NQ = 4
CHUNK = 128
LANES = 16
TBLK = 8192


def _fuse_tables(tabs_t):
    dim, vocab = tabs_t[0].shape
    grid = (vocab + TBLK - 1) // TBLK

    def body(t0_ref, t1_ref, t2_ref, t3_ref, o_ref):
        eye = jnp.where(
            lax.broadcasted_iota(jnp.int32, (128, 128), 0)
            == lax.broadcasted_iota(jnp.int32, (128, 128), 1),
            jnp.float32(1.0), jnp.float32(0.0))
        for q, t_ref in enumerate((t0_ref, t1_ref, t2_ref, t3_ref)):
            x = t_ref[...]
            for c in range(TBLK // 128):
                o_ref[pl.ds(c * 128, 128), pl.ds(q * dim, dim)] = (
                    lax.dot_general(eye, x[:, c * 128:(c + 1) * 128],
                                    (((1,), (1,)), ((), ())),
                                    preferred_element_type=jnp.float32))

    in_spec = pl.BlockSpec((dim, TBLK), lambda i: (0, i))
    return pl.pallas_call(
        body,
        grid=(grid,),
        in_specs=[in_spec, in_spec, in_spec, in_spec],
        out_specs=pl.BlockSpec((TBLK, NQ * dim), lambda i: (i, 0)),
        out_shape=jax.ShapeDtypeStruct((vocab, NQ * dim), jnp.float32),
    )(*tabs_t)


def _make_sc_kernel(l_seq, b, dim, nc, ns):
    nw = nc * ns
    assert b == nw * CHUNK

    mesh = plsc.VectorSubcoreMesh(core_axis_name="c", subcore_axis_name="s")

    @functools.partial(
        pl.kernel,
        out_type=jax.ShapeDtypeStruct((l_seq, b, dim * NQ), jnp.float32),
        mesh=mesh,
        compiler_params=pltpu.CompilerParams(needs_layout_passes=False),
        scratch_types=[
            pltpu.VMEM((l_seq, CHUNK), jnp.int32),
            pltpu.VMEM((CHUNK,), jnp.int32),
            pltpu.VMEM((CHUNK,), jnp.int32),
            pltpu.VMEM((CHUNK, NQ * dim), jnp.float32),
            pltpu.VMEM((CHUNK, NQ * dim), jnp.float32),
            pltpu.VMEM((CHUNK, dim * NQ), jnp.float32),
            pltpu.SemaphoreType.DMA,
            pltpu.SemaphoreType.DMA,
        ],
    )
    def qembed(xt_ref, tab_ref, out_ref, idx_v, idxg_a, idxg_b,
               g_a, g_b, o_v, sem_a, sem_b):
        wid = lax.axis_index("c") * ns + lax.axis_index("s")
        col0 = wid * CHUNK
        pltpu.sync_copy(xt_ref.at[:, pl.ds(col0, CHUNK)], idx_v)

        iota = lax.iota(jnp.int32, LANES)
        pats = [[iota * NQ + (h * LANES * NQ + q) for h in range(dim // LANES)]
                for q in range(NQ)]
        nj = CHUNK // LANES

        def fire(t, idxg, sem):
            for j in range(nj):
                idxg[pl.ds(j * LANES, LANES)] = idx_v[t, pl.ds(j * LANES, LANES)]
            pltpu.async_copy(tab_ref.at[idxg], g_a, sem)

        def fire_b(t, idxg, sem):
            for j in range(nj):
                idxg[pl.ds(j * LANES, LANES)] = idx_v[t, pl.ds(j * LANES, LANES)]
            pltpu.async_copy(tab_ref.at[idxg], g_b, sem)

        def drain(g_v, sem, t):
            pltpu.make_async_copy(tab_ref.at[idxg_a], g_v, sem).wait()

            def interleave(i, carry2):
                row = jnp.full((LANES,), 0, jnp.int32) + i
                for q in range(NQ):
                    for h in range(dim // LANES):
                        vals = g_v[i, pl.ds(q * dim + h * LANES, LANES)]
                        plsc.store_scatter(o_v, [row, pats[q][h]], vals)
                return carry2

            lax.fori_loop(0, CHUNK, interleave, 0, unroll=2)
            pltpu.sync_copy(o_v, out_ref.at[t, pl.ds(col0, CHUNK), :])

        fire(0, idxg_a, sem_a)

        def do_pair(p, carry):
            ta = p * 2
            fire_b(ta + 1, idxg_b, sem_b)
            drain(g_a, sem_a, ta)

            @pl.when(p < (l_seq // 2) - 1)
            def _():
                fire(ta + 2, idxg_a, sem_a)

            drain(g_b, sem_b, ta + 1)
            return carry

        lax.fori_loop(0, l_seq // 2, do_pair, 0)

    return qembed


@jax.jit
def kernel(x, scalar, vector_i, vector_j, vector_k):
    b, l_seq = x.shape
    vocab, dim = scalar.shape
    info = plsc.get_sparse_core_info()
    k = _make_sc_kernel(l_seq, b, dim, info.num_cores, info.num_subcores)
    fused = _fuse_tables([t.T for t in (scalar, vector_i, vector_j, vector_k)])
    out = k(x.T.astype(jnp.int32), fused)
    return out.reshape(l_seq, b, dim, NQ).transpose(1, 0, 2, 3)
